# Initial kernel scaffold; baseline (speedup 1.0000x reference)
import jax
import jax.numpy as jnp
from jax.experimental import pallas as pl


def kernel(t, pos_embeddings):
    raise NotImplementedError("write your pallas kernel here")


# speedup vs baseline: 2.0745x; 2.0745x over previous
import functools

import jax
import jax.numpy as jnp
from jax import lax
from jax.experimental import pallas as pl
from jax.experimental.pallas import tpu as pltpu
from jax.experimental.pallas import tpu_sc as plsc

_B = 16384
_V = 8192
_D = 1024
_NC = 2
_NS = 16
_NW = _NC * _NS
_BPW = _B // _NW
_C = 32
_NCHUNK = _BPW // _C


def _sc_gather(table, t):
    mesh = plsc.VectorSubcoreMesh(
        core_axis_name="c", subcore_axis_name="s",
        num_cores=_NC, num_subcores=_NS,
    )

    @functools.partial(
        pl.kernel,
        out_type=jax.ShapeDtypeStruct((_B, _D), jnp.float32),
        mesh=mesh,
        scratch_types=[
            pltpu.VMEM((_BPW,), jnp.int32),
            pltpu.VMEM((2, _C, _D), jnp.float32),
            pltpu.SemaphoreType.DMA,
            pltpu.SemaphoreType.DMA,
        ],
    )
    def body(table_hbm, t_hbm, out_hbm, idx_v, rows_v, sem_r, sem_w):
        wid = lax.axis_index("s") * _NC + lax.axis_index("c")
        base = wid * _BPW
        pltpu.sync_copy(t_hbm.at[pl.ds(base, _BPW)], idx_v)

        def gather(g, buf):
            return pltpu.make_async_copy(
                table_hbm.at[idx_v.at[pl.ds(g * _C, _C)]],
                rows_v.at[buf],
                sem_r,
            )

        def write(g, buf):
            return pltpu.make_async_copy(
                rows_v.at[buf],
                out_hbm.at[pl.ds(base + g * _C, _C)],
                sem_w,
            )

        gather(0, 0).start()
        for g in range(_NCHUNK):
            buf = g % 2
            if g + 1 < _NCHUNK:
                if g >= 1:
                    write(g - 1, (g + 1) % 2).wait()
                gather(g + 1, (g + 1) % 2).start()
            gather(g, buf).wait()
            write(g, buf).start()
        write(_NCHUNK - 2, _NCHUNK % 2).wait()
        write(_NCHUNK - 1, (_NCHUNK - 1) % 2).wait()

    return body(table, t)


def kernel(t, pos_embeddings):
    return _sc_gather(pos_embeddings, t.astype(jnp.int32))
